# baseline (speedup 1.0000x reference)
import jax
import jax.numpy as jnp
from jax import lax
from jax.experimental import pallas as pl
from jax.experimental.pallas import tpu as pltpu
from jax.experimental.pallas import tpu_sc as plsc

B = 16384
D = 64
NC = 2
NS = 16
NW = NC * NS
BPW = B // NW
CH = 128
NCH = BPW // CH
HALF = 2
RB = 512


def _sc_body(head_hbm, rel_hbm, tail_hbm, ent_hbm, relemb_hbm,
             oh_hbm, or_hbm, ot_hbm, hidx, ridx, tidx, hrow, rrow, trow, sem):
    wid = lax.axis_index("s") * NC + lax.axis_index("c")
    base = wid * BPW
    rowblk = wid * NCH

    pltpu.sync_copy(head_hbm.at[pl.ds(rowblk, NCH)], hidx)
    pltpu.sync_copy(rel_hbm.at[pl.ds(rowblk, NCH)], ridx)
    pltpu.sync_copy(tail_hbm.at[pl.ds(rowblk, NCH)], tidx)

    for h in range(HALF):
        copies = []
        for jj in range(NCH // HALF):
            j = h * (NCH // HALF) + jj
            rows = pl.ds(jj * CH, CH)
            copies.append(pltpu.async_copy(
                ent_hbm.at[hidx.at[j]], hrow.at[rows], sem))
            copies.append(pltpu.async_copy(
                relemb_hbm.at[ridx.at[j]], rrow.at[rows], sem))
            copies.append(pltpu.async_copy(
                ent_hbm.at[tidx.at[j]], trow.at[rows], sem))
        for c in copies:
            c.wait()
        hb = BPW // HALF
        out_rows = pl.ds(base + h * hb, hb)
        pltpu.sync_copy(hrow, oh_hbm.at[out_rows])
        pltpu.sync_copy(rrow, or_hbm.at[out_rows])
        pltpu.sync_copy(trow, ot_hbm.at[out_rows])


def _select_concat_body(h_ref, r_ref, t_ref, par_ref, o_ref):
    par = par_ref[...]

    def half(blk, p):
        lo = blk[:, :D]
        hi = blk[:, D:]
        return jnp.where(par[:, p:p + 1] == 1, hi, lo)

    o_ref[...] = jnp.concatenate(
        [half(h_ref[...], 0), half(r_ref[...], 1), half(t_ref[...], 2)],
        axis=-1)


@jax.jit
def _lookup(head2, rel2, tail2, par, ent2, relemb2):
    gather = pl.kernel(
        _sc_body,
        mesh=plsc.VectorSubcoreMesh(core_axis_name="c", subcore_axis_name="s"),
        out_type=(
            jax.ShapeDtypeStruct((B, 2 * D), jnp.float32),
            jax.ShapeDtypeStruct((B, 2 * D), jnp.float32),
            jax.ShapeDtypeStruct((B, 2 * D), jnp.float32),
        ),
        scratch_types=[
            pltpu.VMEM((NCH, CH), jnp.int32),
            pltpu.VMEM((NCH, CH), jnp.int32),
            pltpu.VMEM((NCH, CH), jnp.int32),
            pltpu.VMEM((BPW // HALF, 2 * D), jnp.float32),
            pltpu.VMEM((BPW // HALF, 2 * D), jnp.float32),
            pltpu.VMEM((BPW // HALF, 2 * D), jnp.float32),
            pltpu.SemaphoreType.DMA,
        ],
    )
    h, r, t = gather(head2, rel2, tail2, ent2, relemb2)

    concat = pl.pallas_call(
        _select_concat_body,
        grid=(B // RB,),
        in_specs=[
            pl.BlockSpec((RB, 2 * D), lambda i: (i, 0)),
            pl.BlockSpec((RB, 2 * D), lambda i: (i, 0)),
            pl.BlockSpec((RB, 2 * D), lambda i: (i, 0)),
            pl.BlockSpec((RB, 3), lambda i: (i, 0)),
        ],
        out_specs=pl.BlockSpec((RB, 3 * D), lambda i: (i, 0)),
        out_shape=jax.ShapeDtypeStruct((B, 3 * D), jnp.float32),
    )
    return concat(h, r, t, par)


def kernel(head, rel, tail, ent_embeddings, rel_embeddings):
    ent2 = jnp.concatenate(
        [ent_embeddings[0::2], ent_embeddings[1::2]], axis=1)
    relemb2 = jnp.concatenate(
        [rel_embeddings[0::2], rel_embeddings[1::2]], axis=1)
    head2 = (head >> 1).reshape(B // CH, CH)
    rel2 = (rel >> 1).reshape(B // CH, CH)
    tail2 = (tail >> 1).reshape(B // CH, CH)
    par = jnp.stack([head & 1, rel & 1, tail & 1], axis=1)
    return _lookup(head2, rel2, tail2, par, ent2, relemb2)


# speedup vs baseline: 20.5769x; 20.5769x over previous
import jax
import jax.numpy as jnp
from jax import lax
from jax.experimental import pallas as pl
from jax.experimental.pallas import tpu as pltpu
from jax.experimental.pallas import tpu_sc as plsc

B = 16384
D = 64
NC = 2
NS = 16
NW = NC * NS
BPW = B // NW
L = 16
GRP = BPW // L
WIN = 128
GPW = WIN // L
NWIN = BPW // WIN
RCH = 128
NRCH = BPW // RCH
RB = 2048


def _ent_window(idx_v, tab, rows, blkbuf, out_hbm, base, w, sem):
    prev = None
    for g in range(GPW):
        off = pl.multiple_of(w * WIN + g * L, 8)
        v = idx_v[pl.ds(off, L)]
        copies = [
            pltpu.async_copy(
                tab.at[pl.ds(v[k], 1)],
                rows.at[pl.ds(g * L + k, 1)], sem)
            for k in range(L)
        ]
        if prev is not None:
            for c in prev:
                c.wait()
        prev = copies
    for c in prev:
        c.wait()
    out_off = pl.multiple_of(base + w * WIN, 8)
    pltpu.sync_copy(rows, out_hbm.at[pl.ds(out_off, WIN)])


def _sc_body(head_hbm, tail_hbm, rp2_hbm, ent3_hbm, relp_hbm,
             oh_hbm, orp_hbm, ot_hbm,
             hidx, tidx, ridx, blkbuf, rows, relrow, sem, rsem):
    wid = lax.axis_index("s") * NC + lax.axis_index("c")
    base = pl.multiple_of(wid * BPW, 8)
    rowblk = wid * NRCH

    pltpu.sync_copy(head_hbm.at[pl.ds(base, BPW)], hidx)
    pltpu.sync_copy(tail_hbm.at[pl.ds(base, BPW)], tidx)
    pltpu.sync_copy(rp2_hbm.at[pl.ds(rowblk, NRCH)], ridx)

    rel_copies = [
        pltpu.async_copy(
            relp_hbm.at[ridx.at[j]], relrow.at[pl.ds(j * RCH, RCH)], rsem)
        for j in range(NRCH)
    ]

    def hwin(w, _):
        _ent_window(hidx, ent3_hbm, rows, blkbuf, oh_hbm, base, w, sem)
        return 0

    def twin(w, _):
        _ent_window(tidx, ent3_hbm, rows, blkbuf, ot_hbm, base, w, sem)
        return 0

    lax.fori_loop(0, NWIN, hwin, 0, unroll=False)
    lax.fori_loop(0, NWIN, twin, 0, unroll=False)

    for c in rel_copies:
        c.wait()
    pltpu.sync_copy(relrow, orp_hbm.at[pl.ds(base, BPW)])


def _select_concat_body(h_ref, rp_ref, t_ref, par_ref, o_ref):
    rp = rp_ref[...]
    r = jnp.where(par_ref[...] == 1, rp[:, D:], rp[:, :D])
    o_ref[...] = jnp.concatenate([h_ref[...], r, t_ref[...]], axis=-1)


@jax.jit
def _lookup(head, rel2, tail, par, ent3, relp):
    gather = pl.kernel(
        _sc_body,
        mesh=plsc.VectorSubcoreMesh(core_axis_name="c", subcore_axis_name="s"),
        out_type=(
            jax.ShapeDtypeStruct((B, D), jnp.float32),
            jax.ShapeDtypeStruct((B, 2 * D), jnp.float32),
            jax.ShapeDtypeStruct((B, D), jnp.float32),
        ),
        scratch_types=[
            pltpu.VMEM((BPW,), jnp.int32),
            pltpu.VMEM((BPW,), jnp.int32),
            pltpu.VMEM((NRCH, RCH), jnp.int32),
            pltpu.VMEM((L, 8, D), jnp.float32),
            pltpu.VMEM((WIN, D), jnp.float32),
            pltpu.VMEM((BPW, 2 * D), jnp.float32),
            pltpu.SemaphoreType.DMA,
            pltpu.SemaphoreType.DMA,
        ],
        compiler_params=pltpu.CompilerParams(needs_layout_passes=False),
    )
    h, rp, t = gather(head, tail, rel2, ent3, relp)

    concat = pl.pallas_call(
        _select_concat_body,
        grid=(B // RB,),
        in_specs=[
            pl.BlockSpec((RB, D), lambda i: (i, 0)),
            pl.BlockSpec((RB, 2 * D), lambda i: (i, 0)),
            pl.BlockSpec((RB, D), lambda i: (i, 0)),
            pl.BlockSpec((RB, 1), lambda i: (i, 0)),
        ],
        out_specs=pl.BlockSpec((RB, 3 * D), lambda i: (i, 0)),
        out_shape=jax.ShapeDtypeStruct((B, 3 * D), jnp.float32),
    )
    return concat(h, rp, t, par)


def kernel(head, rel, tail, ent_embeddings, rel_embeddings):
    ent3 = ent_embeddings
    relp = rel_embeddings.reshape(-1, 2 * D)
    rel2 = (rel >> 1).reshape(B // RCH, RCH)
    par = (rel & 1).reshape(B, 1)
    return _lookup(head, rel2, tail, par, ent3, relp)


# speedup vs baseline: 23.1700x; 1.1260x over previous
import jax
import jax.numpy as jnp
from jax import lax
from jax.experimental import pallas as pl
from jax.experimental.pallas import tpu as pltpu
from jax.experimental.pallas import tpu_sc as plsc

B = 16384
D = 64
NC = 2
NS = 16
NW = NC * NS
BPW = B // NW
L = 16
GRP = BPW // L
WIN = 128
GPW = WIN // L
NWIN = BPW // WIN
RCH = 128
NRCH = BPW // RCH
RB = 2048


def _ent_window(idx_v, tab3, rows, blkbuf, out_hbm, base, w, sem):
    lane = lax.iota(jnp.int32, L)

    def select(g, r, slot):
        dst_row = g * L + lane
        for p in range(D):
            pvec = jnp.full((L,), p, jnp.int32)
            vals = plsc.load_gather(blkbuf.at[slot], [lane, r, pvec])
            plsc.store_scatter(rows, [dst_row, pvec], vals)

    prev = None
    for g in range(GPW):
        off = pl.multiple_of(w * WIN + g * L, 8)
        v = idx_v[pl.ds(off, L)]
        blk = lax.shift_right_logical(v, 3)
        r = lax.bitwise_and(v, 7)
        slot = g % 2
        copies = [
            pltpu.async_copy(tab3.at[blk[k]], blkbuf.at[slot, k], sem)
            for k in range(L)
        ]
        if prev is not None:
            pg, pr, pcopies = prev
            for c in pcopies:
                c.wait()
            select(pg, pr, pg % 2)
        prev = (g, r, copies)
    pg, pr, pcopies = prev
    for c in pcopies:
        c.wait()
    select(pg, pr, pg % 2)
    out_off = pl.multiple_of(base + w * WIN, 8)
    pltpu.sync_copy(rows, out_hbm.at[pl.ds(out_off, WIN)])


def _sc_body(head_hbm, tail_hbm, rp2_hbm, ent3_hbm, relp_hbm,
             oh_hbm, orp_hbm, ot_hbm,
             hidx, tidx, ridx, blkbuf, rows, relrow, sem, rsem):
    wid = lax.axis_index("s") * NC + lax.axis_index("c")
    base = pl.multiple_of(wid * BPW, 8)
    rowblk = wid * NRCH

    pltpu.sync_copy(head_hbm.at[pl.ds(base, BPW)], hidx)
    pltpu.sync_copy(tail_hbm.at[pl.ds(base, BPW)], tidx)
    pltpu.sync_copy(rp2_hbm.at[pl.ds(rowblk, NRCH)], ridx)

    rel_copies = [
        pltpu.async_copy(
            relp_hbm.at[ridx.at[j]], relrow.at[pl.ds(j * RCH, RCH)], rsem)
        for j in range(NRCH)
    ]

    def hwin(w, _):
        _ent_window(hidx, ent3_hbm, rows, blkbuf, oh_hbm, base, w, sem)
        return 0

    def twin(w, _):
        _ent_window(tidx, ent3_hbm, rows, blkbuf, ot_hbm, base, w, sem)
        return 0

    lax.fori_loop(0, NWIN, hwin, 0, unroll=False)
    lax.fori_loop(0, NWIN, twin, 0, unroll=False)

    for c in rel_copies:
        c.wait()
    pltpu.sync_copy(relrow, orp_hbm.at[pl.ds(base, BPW)])


def _select_concat_body(h_ref, rp_ref, t_ref, par_ref, o_ref):
    rp = rp_ref[...]
    r = jnp.where(par_ref[...] == 1, rp[:, D:], rp[:, :D])
    o_ref[...] = jnp.concatenate([h_ref[...], r, t_ref[...]], axis=-1)


@jax.jit
def _lookup(head, rel2, tail, par, ent3, relp):
    gather = pl.kernel(
        _sc_body,
        mesh=plsc.VectorSubcoreMesh(core_axis_name="c", subcore_axis_name="s"),
        out_type=(
            jax.ShapeDtypeStruct((B, D), jnp.float32),
            jax.ShapeDtypeStruct((B, 2 * D), jnp.float32),
            jax.ShapeDtypeStruct((B, D), jnp.float32),
        ),
        scratch_types=[
            pltpu.VMEM((BPW,), jnp.int32),
            pltpu.VMEM((BPW,), jnp.int32),
            pltpu.VMEM((NRCH, RCH), jnp.int32),
            pltpu.VMEM((2, L, 8, D), jnp.float32),
            pltpu.VMEM((WIN, D), jnp.float32),
            pltpu.VMEM((BPW, 2 * D), jnp.float32),
            pltpu.SemaphoreType.DMA,
            pltpu.SemaphoreType.DMA,
        ],
        compiler_params=pltpu.CompilerParams(needs_layout_passes=False),
    )
    h, rp, t = gather(head, tail, rel2, ent3, relp)

    concat = pl.pallas_call(
        _select_concat_body,
        grid=(B // RB,),
        in_specs=[
            pl.BlockSpec((RB, D), lambda i: (i, 0)),
            pl.BlockSpec((RB, 2 * D), lambda i: (i, 0)),
            pl.BlockSpec((RB, D), lambda i: (i, 0)),
            pl.BlockSpec((RB, 1), lambda i: (i, 0)),
        ],
        out_specs=pl.BlockSpec((RB, 3 * D), lambda i: (i, 0)),
        out_shape=jax.ShapeDtypeStruct((B, 3 * D), jnp.float32),
    )
    return concat(h, rp, t, par)


def kernel(head, rel, tail, ent_embeddings, rel_embeddings):
    ent3 = ent_embeddings.reshape(-1, 8, D)
    relp = rel_embeddings.reshape(-1, 2 * D)
    rel2 = (rel >> 1).reshape(B // RCH, RCH)
    par = (rel & 1).reshape(B, 1)
    return _lookup(head, rel2, tail, par, ent3, relp)
